# initial kernel scaffold (unmeasured)
import jax
import jax.numpy as jnp
from jax import lax
from jax.experimental import pallas as pl
from jax.experimental.pallas import tpu as pltpu


def kernel(
    x,
):
    def body(*refs):
        pass

    out_shape = jax.ShapeDtypeStruct(..., jnp.float32)
    return pl.pallas_call(body, out_shape=out_shape)(...)


# baseline (device time: 54701 ns/iter reference)
import jax
import jax.numpy as jnp
from jax import lax
from jax.experimental import pallas as pl
from jax.experimental.pallas import tpu as pltpu

M = 2048
N = 1024
H = N // 2


def kernel(x):
    def body(x_ref, out_ref, comm_ref, send_sem, recv_sem):
        my_x = lax.axis_index("x")
        my_y = lax.axis_index("y")
        my_z = lax.axis_index("z")
        peer = (1 - my_x, my_y, my_z)

        barrier_sem = pltpu.get_barrier_semaphore()
        pl.semaphore_signal(
            barrier_sem, inc=1, device_id=peer, device_id_type=pl.DeviceIdType.MESH
        )
        pl.semaphore_wait(barrier_sem, 1)

        def exchange(my_lo, peer_lo):
            rdma = pltpu.make_async_remote_copy(
                src_ref=x_ref.at[0, :, pl.ds(peer_lo, H)],
                dst_ref=comm_ref,
                send_sem=send_sem,
                recv_sem=recv_sem,
                device_id=peer,
                device_id_type=pl.DeviceIdType.MESH,
            )
            rdma.start()
            rdma.wait()
            out_ref[:, :] = x_ref[0, :, pl.ds(my_lo, H)] + comm_ref[:, :]

        @pl.when(my_x == 0)
        def _():
            exchange(0, H)

        @pl.when(my_x == 1)
        def _():
            exchange(H, 0)

    return pl.pallas_call(
        body,
        out_shape=jax.ShapeDtypeStruct((M, H), jnp.float32),
        in_specs=[pl.BlockSpec(memory_space=pltpu.VMEM)],
        out_specs=pl.BlockSpec(memory_space=pltpu.VMEM),
        scratch_shapes=[
            pltpu.VMEM((M, H), jnp.float32),
            pltpu.SemaphoreType.DMA,
            pltpu.SemaphoreType.DMA,
        ],
        compiler_params=pltpu.CompilerParams(collective_id=0),
    )(x)


# device time: 36162 ns/iter; 1.5127x vs baseline; 1.5127x over previous
import jax
import jax.numpy as jnp
from jax import lax
from jax.experimental import pallas as pl
from jax.experimental.pallas import tpu as pltpu

M = 2048
N = 1024
H = N // 2
RY = M // 2
K = 16
CH = RY // K


def kernel(x):
    def body(x_ref, out_ref, xrecv, sx, rx, sy, ry_sem):
        my_x = lax.axis_index("x")
        my_y = lax.axis_index("y")
        my_z = lax.axis_index("z")
        x_peer = (1 - my_x, my_y, my_z)
        y_nbr = (my_x, 1 - my_y, my_z)

        barrier_sem = pltpu.get_barrier_semaphore()
        for nbr in (x_peer, y_nbr):
            pl.semaphore_signal(
                barrier_sem, inc=1, device_id=nbr, device_id_type=pl.DeviceIdType.MESH
            )
        pl.semaphore_wait(barrier_sem, 2)

        my_row = my_y * RY

        def run(my_lo, peer_lo):
            x_rdmas = []
            for k in range(K):
                rows = pl.ds(my_row + k * CH, CH)
                rdma = pltpu.make_async_remote_copy(
                    src_ref=x_ref.at[0, rows, peer_lo : peer_lo + H],
                    dst_ref=xrecv.at[k],
                    send_sem=sx.at[k],
                    recv_sem=rx.at[k],
                    device_id=x_peer,
                    device_id_type=pl.DeviceIdType.MESH,
                )
                rdma.start()
                x_rdmas.append(rdma)

            y_rdmas = []
            for k in range(K):
                rows = pl.ds(my_row + k * CH, CH)
                x_rdmas[k].wait_recv()
                out_ref[rows, :] = x_ref[0, rows, my_lo : my_lo + H] + xrecv[k]
                rdma = pltpu.make_async_remote_copy(
                    src_ref=out_ref.at[rows],
                    dst_ref=out_ref.at[rows],
                    send_sem=sy.at[k],
                    recv_sem=ry_sem.at[k],
                    device_id=y_nbr,
                    device_id_type=pl.DeviceIdType.MESH,
                )
                rdma.start()
                y_rdmas.append(rdma)

            for k in range(K):
                y_rdmas[k].wait_recv()
                y_rdmas[k].wait_send()
                x_rdmas[k].wait_send()

        @pl.when(my_x == 0)
        def _():
            run(0, H)

        @pl.when(my_x == 1)
        def _():
            run(H, 0)

    return pl.pallas_call(
        body,
        out_shape=jax.ShapeDtypeStruct((M, H), jnp.float32),
        in_specs=[pl.BlockSpec(memory_space=pltpu.VMEM)],
        out_specs=pl.BlockSpec(memory_space=pltpu.VMEM),
        scratch_shapes=[
            pltpu.VMEM((K, CH, H), jnp.float32),
            pltpu.SemaphoreType.DMA((K,)),
            pltpu.SemaphoreType.DMA((K,)),
            pltpu.SemaphoreType.DMA((K,)),
            pltpu.SemaphoreType.DMA((K,)),
        ],
        compiler_params=pltpu.CompilerParams(collective_id=0),
    )(x)


# device time: 35764 ns/iter; 1.5295x vs baseline; 1.0111x over previous
import jax
import jax.numpy as jnp
from jax import lax
from jax.experimental import pallas as pl
from jax.experimental.pallas import tpu as pltpu

M = 2048
N = 1024
H = N // 2
CH = 64
KX = 11
KZ = 10


def kernel(x):
    def body(x_ref, out_ref, xrecv, sx, rx, sy, ry, sz, rz):
        my_x = lax.axis_index("x")
        my_y = lax.axis_index("y")
        my_z = lax.axis_index("z")
        x_peer = (1 - my_x, my_y, my_z)
        y_nbr = (my_x, 1 - my_y, my_z)
        z_nbr = (my_x, my_y, 1 - my_z)

        barrier_sem = pltpu.get_barrier_semaphore()
        for nbr in (x_peer, y_nbr, z_nbr):
            pl.semaphore_signal(
                barrier_sem, inc=1, device_id=nbr, device_id_type=pl.DeviceIdType.MESH
            )
        pl.semaphore_wait(barrier_sem, 3)

        e_row = (22 * my_z + 5 * my_y) * CH
        c_row = (10 + 6 * my_y) * CH
        e2_row = (22 * my_z + 5 * (1 - my_y)) * CH
        c2_row = (10 + 6 * (1 - my_y)) * CH

        def unit_row(k):
            return e_row + k * CH if k < 5 else c_row + (k - 5) * CH

        def unit_row_nbr(k):
            return e2_row + k * CH if k < 5 else c2_row + (k - 5) * CH

        def run(my_lo, peer_lo):
            xr = []
            for k in range(KX):
                rows = pl.ds(unit_row(k), CH)
                d = pltpu.make_async_remote_copy(
                    src_ref=x_ref.at[0, rows, peer_lo : peer_lo + H],
                    dst_ref=xrecv.at[k],
                    send_sem=sx.at[k],
                    recv_sem=rx.at[k],
                    device_id=x_peer,
                    device_id_type=pl.DeviceIdType.MESH,
                )
                d.start()
                xr.append(d)

            yr = []
            zr = []
            for k in range(KX):
                rows = pl.ds(unit_row(k), CH)
                xr[k].wait_recv()
                out_ref[rows, :] = x_ref[0, rows, my_lo : my_lo + H] + xrecv[k]
                dy = pltpu.make_async_remote_copy(
                    src_ref=out_ref.at[rows],
                    dst_ref=out_ref.at[rows],
                    send_sem=sy.at[k],
                    recv_sem=ry.at[k],
                    device_id=y_nbr,
                    device_id_type=pl.DeviceIdType.MESH,
                )
                dy.start()
                yr.append(dy)
                if k < 5:
                    dz = pltpu.make_async_remote_copy(
                        src_ref=out_ref.at[rows],
                        dst_ref=out_ref.at[rows],
                        send_sem=sz.at[k],
                        recv_sem=rz.at[k],
                        device_id=z_nbr,
                        device_id_type=pl.DeviceIdType.MESH,
                    )
                    dz.start()
                    zr.append(dz)

            for k in range(KX):
                yr[k].wait_recv()
                if k < 5:
                    rows = pl.ds(unit_row_nbr(k), CH)
                    dz = pltpu.make_async_remote_copy(
                        src_ref=out_ref.at[rows],
                        dst_ref=out_ref.at[rows],
                        send_sem=sz.at[5 + k],
                        recv_sem=rz.at[5 + k],
                        device_id=z_nbr,
                        device_id_type=pl.DeviceIdType.MESH,
                    )
                    dz.start()
                    zr.append(dz)

            for j in range(KZ):
                zr[j].wait_recv()
            for k in range(KX):
                xr[k].wait_send()
                yr[k].wait_send()
            for j in range(KZ):
                zr[j].wait_send()

        @pl.when(my_x == 0)
        def _():
            run(0, H)

        @pl.when(my_x == 1)
        def _():
            run(H, 0)

    return pl.pallas_call(
        body,
        out_shape=jax.ShapeDtypeStruct((M, H), jnp.float32),
        in_specs=[pl.BlockSpec(memory_space=pltpu.VMEM)],
        out_specs=pl.BlockSpec(memory_space=pltpu.VMEM),
        scratch_shapes=[
            pltpu.VMEM((KX, CH, H), jnp.float32),
            pltpu.SemaphoreType.DMA((KX,)),
            pltpu.SemaphoreType.DMA((KX,)),
            pltpu.SemaphoreType.DMA((KX,)),
            pltpu.SemaphoreType.DMA((KX,)),
            pltpu.SemaphoreType.DMA((KZ,)),
            pltpu.SemaphoreType.DMA((KZ,)),
        ],
        compiler_params=pltpu.CompilerParams(collective_id=0),
    )(x)
